# Initial kernel scaffold; baseline (speedup 1.0000x reference)
import jax
import jax.numpy as jnp
from jax.experimental import pallas as pl


def kernel(x, ei_x, ei_y, ei_z, w_x, w_y, w_z, W_sub, W_coef, b_coef):
    raise NotImplementedError("write your pallas kernel here")


# speedup vs baseline: 3.5483x; 3.5483x over previous
import functools

import jax
import jax.numpy as jnp
from jax import lax
from jax.experimental import pallas as pl
from jax.experimental.pallas import tpu as pltpu
from jax.experimental.pallas import tpu_sc as plsc

_N = 10000
_E = 160000
_F = 128
_NC = 2
_NS = 16
_NW = _NC * _NS
_CH = 128
_NCHUNKS = _E // _CH
_CHUNKS_PER_W = _NCHUNKS // _NW
_CHUNK_REM = _NCHUNKS - _NW * _CHUNKS_PER_W
_NP = 10240
_RPT = _NP // _NS
_ZROWS = 128
_LANES = 16

_mesh = plsc.VectorSubcoreMesh(core_axis_name="c", subcore_axis_name="s")


@functools.partial(
    pl.kernel,
    out_type=jax.ShapeDtypeStruct((3, _NC, _NP, _F), jnp.float32),
    mesh=_mesh,
    scratch_types=[
        pltpu.VMEM((_CH,), jnp.int32),
        pltpu.VMEM((_CH,), jnp.int32),
        pltpu.VMEM((_CH,), jnp.float32),
        pltpu.VMEM((_CH, _F), jnp.float32),
        pltpu.VMEM((_ZROWS, _F), jnp.float32),
        pltpu.VMEM_SHARED((_NP, _F), jnp.float32),
        pltpu.SemaphoreType.DMA,
    ],
)
def _sc_spmm(x_hbm, src_hbm, dst_hbm, w_hbm, out_hbm,
             src_v, dst_v, w_v, rows_v, zeros_v, acc_sh, sem):
    cid = lax.axis_index("c")
    sid = lax.axis_index("s")
    wid = sid * _NC + cid
    row0 = sid * _RPT

    zv = jnp.zeros((_LANES,), jnp.float32)

    def _zfill(i, carry):
        for cb in range(_F // _LANES):
            zeros_v[i, pl.ds(cb * _LANES, _LANES)] = zv
        return carry

    lax.fori_loop(0, _ZROWS, _zfill, 0)

    n_my = _CHUNKS_PER_W + (wid < _CHUNK_REM).astype(jnp.int32)

    for k in range(3):
        for r in range(_RPT // _ZROWS):
            pltpu.sync_copy(zeros_v, acc_sh.at[pl.ds(row0 + r * _ZROWS, _ZROWS)])
        plsc.subcore_barrier()

        def chunk_body(j, carry):
            e0 = k * _E + (wid + _NW * j) * _CH
            pltpu.sync_copy(src_hbm.at[pl.ds(e0, _CH)], src_v)
            pltpu.sync_copy(dst_hbm.at[pl.ds(e0, _CH)], dst_v)
            pltpu.sync_copy(w_hbm.at[pl.ds(e0, _CH)], w_v)
            pltpu.async_copy(x_hbm.at[src_v], rows_v, sem).wait()

            def g_body(g, c2):
                w16 = w_v[pl.ds(g * _LANES, _LANES)]
                for l in range(_LANES):
                    wsplat = w16.at[jnp.full((_LANES,), l, jnp.int32)].get(
                        mode="promise_in_bounds")
                    e = g * _LANES + l
                    for cb in range(_F // _LANES):
                        sl = pl.ds(cb * _LANES, _LANES)
                        rows_v[e, sl] = rows_v[e, sl] * wsplat
                return c2

            lax.fori_loop(0, _CH // _LANES, g_body, 0)

            pltpu.sync_copy(rows_v, acc_sh.at[dst_v], add=True)
            return carry

        lax.fori_loop(0, n_my, chunk_body, 0)
        plsc.subcore_barrier()

        pltpu.sync_copy(acc_sh.at[pl.ds(row0, _RPT)],
                        out_hbm.at[k, cid, pl.ds(row0, _RPT)])


_BN = 1000


def _tc_finish_body(p_ref, wsub_ref, wcoef_ref, b_ref, out_ref):
    wsub = wsub_ref[...]
    hs = []
    for k in range(3):
        hk = p_ref[k, 0] + p_ref[k, 1]
        hs.append(lax.dot(hk, wsub, precision=lax.Precision.HIGHEST))
    c = hs[0] * hs[0] + hs[1] * hs[1] + hs[2] * hs[2]
    t = jnp.tanh(
        lax.dot(c, wcoef_ref[...], precision=lax.Precision.HIGHEST) + b_ref[...])
    out_ref[...] = jnp.stack([h * t for h in hs], axis=1)


def _tc_finish(partials, W_sub, W_coef, b_coef):
    return pl.pallas_call(
        _tc_finish_body,
        grid=(_N // _BN,),
        in_specs=[
            pl.BlockSpec((3, _NC, _BN, _F), lambda i: (0, 0, i, 0)),
            pl.BlockSpec((_F, _F), lambda i: (0, 0)),
            pl.BlockSpec((_F, _F), lambda i: (0, 0)),
            pl.BlockSpec((1, _F), lambda i: (0, 0)),
        ],
        out_specs=pl.BlockSpec((_BN, 3, _F), lambda i: (i, 0, 0)),
        out_shape=jax.ShapeDtypeStruct((_N, 3, _F), jnp.float32),
    )(partials, W_sub, W_coef, b_coef.reshape(1, _F))


def kernel(x, ei_x, ei_y, ei_z, w_x, w_y, w_z, W_sub, W_coef, b_coef):
    src = jnp.concatenate([ei_x[1], ei_y[1], ei_z[1]])
    dst = jnp.concatenate([ei_x[0], ei_y[0], ei_z[0]])
    w = jnp.concatenate([w_x, w_y, w_z])
    partials = _sc_spmm(x, src, dst, w)
    return _tc_finish(partials, W_sub, W_coef, b_coef)
